# baseline (speedup 1.0000x reference)
import functools
import math

import jax
from jax import lax
import jax.numpy as jnp
from jax.experimental import pallas as pl
from jax.experimental.pallas import tpu as pltpu
from jax.experimental.pallas import tpu_sc as plsc

_B, _N = 16, 1024
_KMIN, _KMAX = 5, 20
_HEADS = 4
_EPS = 1e-5
_P = _B * _N
_NW = 32
_PW = _P // _NW


def _sc_gather_max20(fea_pad, idx_flat):
    k, cp, ch = _KMAX, 128, 8
    nsteps = _PW // ch

    mesh = plsc.VectorSubcoreMesh(core_axis_name="c", subcore_axis_name="s")

    @functools.partial(
        pl.kernel, mesh=mesh,
        out_type=jax.ShapeDtypeStruct((_P, cp), jnp.float32),
        scratch_types=[
            pltpu.VMEM((80,), jnp.int32),
            pltpu.VMEM((80,), jnp.int32),
            pltpu.VMEM((80, cp), jnp.float32),
            pltpu.VMEM((80, cp), jnp.float32),
            pltpu.VMEM((ch, cp), jnp.float32),
            pltpu.SemaphoreType.DMA,
        ],
    )
    def sc_kernel(fea_hbm, idx_hbm, out_hbm, idx0, idx1, rows0, rows1, outv, sem):
        cc_ = lax.axis_index("c")
        ss_ = lax.axis_index("s")
        w = ss_ * 2 + cc_
        base = w * _PW
        bias = (w // 2) * _N

        def step(t, carry):
            pt = base + t * ch
            pltpu.sync_copy(idx_hbm.at[pl.ds(pt * k, 80)], idx0)
            pltpu.sync_copy(idx_hbm.at[pl.ds(pt * k + 80, 80)], idx1)
            for i in range(5):
                sl = pl.ds(i * 16, 16)
                idx0[sl] = idx0[sl] + bias
                idx1[sl] = idx1[sl] + bias
            cp0 = pltpu.async_copy(fea_hbm.at[idx0], rows0, sem)
            cp1 = pltpu.async_copy(fea_hbm.at[idx1], rows1, sem)
            cp0.wait()
            cp1.wait()
            for p in range(ch):
                rows = rows0 if p < 4 else rows1
                r0 = (p % 4) * k
                for c0 in range(cp // 16):
                    sl = pl.ds(c0 * 16, 16)
                    acc = rows[r0, sl]
                    for j in range(1, k):
                        acc = jnp.maximum(acc, rows[r0 + j, sl])
                    outv[p, sl] = acc
            pltpu.sync_copy(outv, out_hbm.at[pl.ds(pt, ch)])
            return carry

        lax.fori_loop(0, nsteps, step, 0)

    return sc_kernel(fea_pad, idx_flat)


def _knn_body(x_ref, dist_ref, idx_ref):
    p = x_ref[0]
    cmask = jax.lax.broadcasted_iota(jnp.int32, (1, 4), 1) < 3
    p3 = jnp.where(cmask, p, 0.0)
    g = jnp.dot(p3, p3.T, preferred_element_type=jnp.float32)
    sq = jnp.sum(p3 * p3, axis=1, keepdims=True)
    d2 = sq + sq.T - 2.0 * g
    d = jnp.sqrt(jnp.clip(d2, 1e-12, None))
    iota = jax.lax.broadcasted_iota(jnp.int32, (_N, _N), 1)
    dcols, icols = [], []
    for _ in range(_KMAX):
        cur = jnp.min(d, axis=1, keepdims=True)
        am = jnp.min(jnp.where(d == cur, iota, _N), axis=1, keepdims=True)
        dcols.append(cur)
        icols.append(am)
        d = jnp.where(iota == am, jnp.inf, d)
    dist_ref[0] = jnp.concatenate(dcols, axis=1)
    idx_ref[0] = jnp.concatenate(icols, axis=1)


def _knn(xr):
    return pl.pallas_call(
        _knn_body,
        grid=(_B,),
        in_specs=[pl.BlockSpec((1, _N, 4), lambda b: (b, 0, 0))],
        out_specs=[pl.BlockSpec((1, _N, _KMAX), lambda b: (b, 0, 0)),
                   pl.BlockSpec((1, _N, _KMAX), lambda b: (b, 0, 0))],
        out_shape=[jax.ShapeDtypeStruct((_B, _N, _KMAX), jnp.float32),
                   jax.ShapeDtypeStruct((_B, _N, _KMAX), jnp.int32)],
    )(xr)


def _bn_rows(y, g, b, slope):
    m = jnp.mean(y, axis=0, keepdims=True)
    v = jnp.mean(y * y, axis=0, keepdims=True) - m * m
    z = (y - m) / jnp.sqrt(v + _EPS) * g + b
    if slope == 1.0:
        return z
    return jnp.where(z >= 0, z, slope * z)


def _mlp_body(x_ref, w0, b0r, w1, b1r, w2, b2r, w3, b3r,
              g0, be0, g1, be1, g2, be2, g3, be3, out_ref):
    h = jnp.dot(x_ref[...], w0[...].T, preferred_element_type=jnp.float32) + b0r[...]
    h = _bn_rows(h, g0[...], be0[...], 0.0)
    h = jnp.dot(h, w1[...].T, preferred_element_type=jnp.float32) + b1r[...]
    h = _bn_rows(h, g1[...], be1[...], 0.0)
    h = jnp.dot(h, w2[...].T, preferred_element_type=jnp.float32) + b2r[...]
    h = _bn_rows(h, g2[...], be2[...], 0.0)
    h = jnp.dot(h, w3[...].T, preferred_element_type=jnp.float32) + b3r[...]
    out_ref[...] = _bn_rows(h, g3[...], be3[...], 0.0)


def _mlp(xf, params):
    return pl.pallas_call(
        _mlp_body,
        out_shape=jax.ShapeDtypeStruct((_B * _N, 128), jnp.float32),
    )(xf, *params)


def _gather_conv_body(fea_ref, idx_ref, dist_ref, w_ref, out_ref, *, k):
    f = fea_ref[0]
    idx = idx_ref[0]
    iota = jax.lax.broadcasted_iota(jnp.int32, (_N, _N), 1)
    feat = jnp.full(f.shape, -jnp.inf, f.dtype)
    for j in range(k):
        oh = (idx[:, j:j + 1] == iota).astype(jnp.float32)
        gj = jnp.dot(oh, f, preferred_element_type=jnp.float32)
        feat = jnp.maximum(feat, gj)
    dmax = jnp.max(dist_ref[0], axis=1, keepdims=True)
    edge = jnp.concatenate([f, feat - f, dmax], axis=1)
    out_ref[0] = jnp.dot(edge, w_ref[...].T, preferred_element_type=jnp.float32)


def _gather_conv(fea, idx, dist, w, k):
    c = fea.shape[-1]
    co = w.shape[0]
    return pl.pallas_call(
        functools.partial(_gather_conv_body, k=k),
        grid=(_B,),
        in_specs=[pl.BlockSpec((1, _N, c), lambda b: (b, 0, 0)),
                  pl.BlockSpec((1, _N, k), lambda b: (b, 0, 0)),
                  pl.BlockSpec((1, _N, k), lambda b: (b, 0, 0)),
                  pl.BlockSpec(w.shape, lambda b: (0, 0))],
        out_specs=pl.BlockSpec((1, _N, co), lambda b: (b, 0, 0)),
        out_shape=jax.ShapeDtypeStruct((_B, _N, co), jnp.float32),
    )(fea, idx, dist, w)


def _edge_conv_body(fea_ref, feat_ref, dist_ref, w_ref, out_ref, *, c):
    f = fea_ref[0]
    ft = feat_ref[0][:, :c]
    dmax = jnp.max(dist_ref[0], axis=1, keepdims=True)
    edge = jnp.concatenate([f, ft - f, dmax], axis=1)
    out_ref[0] = jnp.dot(edge, w_ref[...].T, preferred_element_type=jnp.float32)


def _edge_conv(fea, feat, dist, w, k):
    c = fea.shape[-1]
    cp = feat.shape[-1]
    co = w.shape[0]
    return pl.pallas_call(
        functools.partial(_edge_conv_body, c=c),
        grid=(_B,),
        in_specs=[pl.BlockSpec((1, _N, c), lambda b: (b, 0, 0)),
                  pl.BlockSpec((1, _N, cp), lambda b: (b, 0, 0)),
                  pl.BlockSpec((1, _N, k), lambda b: (b, 0, 0)),
                  pl.BlockSpec(w.shape, lambda b: (0, 0))],
        out_specs=pl.BlockSpec((1, _N, co), lambda b: (b, 0, 0)),
        out_shape=jax.ShapeDtypeStruct((_B, _N, co), jnp.float32),
    )(fea, feat, dist, w)


def _gather_rcs_body(fea_ref, idx_ref, dist_ref, w_ref, rw_ref, rb_ref,
                     out_ref, rcs_ref, *, k):
    f = fea_ref[0]
    idx = idx_ref[0]
    iota = jax.lax.broadcasted_iota(jnp.int32, (_N, _N), 1)
    feat = jnp.full(f.shape, -jnp.inf, f.dtype)
    cols = [f[:, 3:4]]
    for j in range(k):
        oh = (idx[:, j:j + 1] == iota).astype(jnp.float32)
        gj = jnp.dot(oh, f, preferred_element_type=jnp.float32)
        feat = jnp.maximum(feat, gj)
        cols.append(gj[:, 3:4])
    rcs = jnp.concatenate(cols, axis=1)
    rcs_ref[0] = jnp.dot(rcs, rw_ref[...].T,
                         preferred_element_type=jnp.float32) + rb_ref[...]
    dmax = jnp.max(dist_ref[0], axis=1, keepdims=True)
    base = f[:, :3]
    edge = jnp.concatenate([base, feat[:, :3] - base, dmax], axis=1)
    out_ref[0] = jnp.dot(edge, w_ref[...].T, preferred_element_type=jnp.float32)


def _gather_rcs(fea, idx, dist, w, rw, rb, k):
    return pl.pallas_call(
        functools.partial(_gather_rcs_body, k=k),
        grid=(_B,),
        in_specs=[pl.BlockSpec((1, _N, 4), lambda b: (b, 0, 0)),
                  pl.BlockSpec((1, _N, k), lambda b: (b, 0, 0)),
                  pl.BlockSpec((1, _N, k), lambda b: (b, 0, 0)),
                  pl.BlockSpec(w.shape, lambda b: (0, 0)),
                  pl.BlockSpec(rw.shape, lambda b: (0, 0)),
                  pl.BlockSpec(rb.shape, lambda b: (0, 0))],
        out_specs=[pl.BlockSpec((1, _N, w.shape[0]), lambda b: (b, 0, 0)),
                   pl.BlockSpec((1, _N, rw.shape[0]), lambda b: (b, 0, 0))],
        out_shape=[jax.ShapeDtypeStruct((_B, _N, w.shape[0]), jnp.float32),
                   jax.ShapeDtypeStruct((_B, _N, rw.shape[0]), jnp.float32)],
    )(fea, idx, dist, w, rw, rb)


def _bn_act_body(y_ref, g_ref, b_ref, o_ref, *, slope):
    o_ref[...] = _bn_rows(y_ref[...], g_ref[...], b_ref[...], slope)


def _bn_act(y, g, b, slope):
    return pl.pallas_call(
        functools.partial(_bn_act_body, slope=slope),
        out_shape=jax.ShapeDtypeStruct(y.shape, jnp.float32),
    )(y, g, b)


def _mix_body(y5_ref, g5, b5, y6_ref, g6, b6, sig_ref, o_ref):
    z5 = _bn_rows(y5_ref[...], g5[...], b5[...], 0.2)
    z6 = _bn_rows(y6_ref[...], g6[...], b6[...], 0.2)
    s = sig_ref[0, 0]
    o_ref[...] = s * z5 + (1.0 - s) * z6


def _mix(y5, g5, b5, y6, g6, b6, sigma):
    return pl.pallas_call(
        _mix_body,
        out_shape=jax.ShapeDtypeStruct(y5.shape, jnp.float32),
    )(y5, g5, b5, y6, g6, b6, sigma)


def _mha_body(h_ref, x3_ref, wq, bq, wk, bk, wv, bv, wo, bo, out_ref):
    hq = h_ref[0]
    ctx = x3_ref[0]
    q = jnp.dot(hq, wq[...].T, preferred_element_type=jnp.float32) + bq[...]
    k_ = jnp.dot(ctx, wk[...].T, preferred_element_type=jnp.float32) + bk[...]
    v = jnp.dot(ctx, wv[...].T, preferred_element_type=jnp.float32) + bv[...]
    dh = 128 // _HEADS
    scale = 1.0 / math.sqrt(1.0 * dh)
    outs = []
    for hh in range(_HEADS):
        qh = q[:, hh * dh:(hh + 1) * dh]
        kh = k_[:, hh * dh:(hh + 1) * dh]
        vh = v[:, hh * dh:(hh + 1) * dh]
        s = jnp.dot(qh, kh.T, preferred_element_type=jnp.float32) * scale
        s = s - jnp.max(s, axis=1, keepdims=True)
        e = jnp.exp(s)
        p = e / jnp.sum(e, axis=1, keepdims=True)
        outs.append(jnp.dot(p, vh, preferred_element_type=jnp.float32))
    o = jnp.concatenate(outs, axis=1)
    out_ref[0] = jnp.dot(o, wo[...].T, preferred_element_type=jnp.float32) + bo[...]


def _mha(h, x3, wq, bq, wk, bk, wv, bv, wo, bo):
    wspec = [pl.BlockSpec(a.shape, lambda b: (0,) * a.ndim)
             for a in (wq, bq, wk, bk, wv, bv, wo, bo)]
    return pl.pallas_call(
        _mha_body,
        grid=(_B,),
        in_specs=[pl.BlockSpec((1, _N, 128), lambda b: (b, 0, 0)),
                  pl.BlockSpec((1, _N, 128), lambda b: (b, 0, 0))] + wspec,
        out_specs=pl.BlockSpec((1, _N, 128), lambda b: (b, 0, 0)),
        out_shape=jax.ShapeDtypeStruct((_B, _N, 128), jnp.float32),
    )(h, x3, wq, bq, wk, bk, wv, bv, wo, bo)


def _cew_body(h_ref, a_ref, wA, wB, ceg, ceb, w4, b4r, g4, be4, out_ref):
    y = (jnp.dot(h_ref[...], wA[...].T, preferred_element_type=jnp.float32)
         + jnp.dot(a_ref[...], wB[...].T, preferred_element_type=jnp.float32))
    x3e = _bn_rows(y, ceg[...], ceb[...], 0.0)
    y2 = jnp.dot(x3e, w4[...].T, preferred_element_type=jnp.float32) + b4r[...]
    out_ref[...] = _bn_rows(y2, g4[...], be4[...], 1.0)


def _cew(h, a, wA, wB, ceg, ceb, w4, b4r, g4, be4):
    return pl.pallas_call(
        _cew_body,
        out_shape=jax.ShapeDtypeStruct((_B * _N, 128), jnp.float32),
    )(h, a, wA, wB, ceg, ceb, w4, b4r, g4, be4)


def _fattn_body(f1_ref, f2_ref, o_ref, gam_ref, out_ref):
    f1 = f1_ref[0]
    f2 = f2_ref[0]
    ob = o_ref[0]
    s = jnp.dot(f1, f2.T, preferred_element_type=jnp.float32) / math.sqrt(32.0)
    s = s - jnp.max(s, axis=1, keepdims=True)
    e = jnp.exp(s)
    p = e / jnp.sum(e, axis=1, keepdims=True)
    enh = jnp.dot(p, ob, preferred_element_type=jnp.float32)
    out_ref[0] = ob + gam_ref[0, 0] * enh


def _fattn(f1, f2, o, gamma):
    return pl.pallas_call(
        _fattn_body,
        grid=(_B,),
        in_specs=[pl.BlockSpec((1, _N, 32), lambda b: (b, 0, 0)),
                  pl.BlockSpec((1, _N, 32), lambda b: (b, 0, 0)),
                  pl.BlockSpec((1, _N, 128), lambda b: (b, 0, 0)),
                  pl.BlockSpec((1, 1), lambda b: (0, 0))],
        out_specs=pl.BlockSpec((1, _N, 128), lambda b: (b, 0, 0)),
        out_shape=jax.ShapeDtypeStruct((_B, _N, 128), jnp.float32),
    )(f1, f2, o, gamma)


def kernel(x, cw0, cb0, cw1, cb1, cw2, cb2, cw3, cb3, cw4, cb4, g0, b0, g1, b1, g2, b2, g3, b3, g4, b4, dw0, dg0, db0, dw1, dg1, db1, dw2, dg2, db2, dw3, dg3, db3, dw4, dg4, db4, dw5, dg5, db5, cew, ceg, ceb, Wq, bq, Wk, bk, Wv, bv, Wo, bo, sigma, gamma_p, rw1, rb1, rw2, rb2):
    xr = x[:, 0]
    xf = xr.reshape(_B * _N, 4)

    r2 = lambda a: a.reshape(1, -1)

    dist20, idx20 = _knn(xr)
    dist5, idx5 = dist20[:, :, :_KMIN], idx20[:, :, :_KMIN]

    h = _mlp(xf, (cw0, r2(cb0), cw1, r2(cb1), cw2, r2(cb2), cw3, r2(cb3),
                  r2(g0), r2(b0), r2(g1), r2(b1), r2(g2), r2(b2), r2(g3), r2(b3)))

    y1, f1 = _gather_rcs(xr, idx5, dist5, dw0, rw1, r2(rb1), _KMIN)
    y2, f2 = _gather_rcs(xr, idx20, dist20, dw1, rw2, r2(rb2), _KMAX)
    z1 = _bn_act(y1.reshape(_B * _N, 64), r2(dg0), r2(db0), 0.2)
    z2 = _bn_act(y2.reshape(_B * _N, 64), r2(dg1), r2(db1), 0.2)
    xg1 = jnp.concatenate([z1.reshape(_B, _N, 64), xr[:, :, :3]], axis=2)
    xg2 = jnp.concatenate([z2.reshape(_B, _N, 64), xr[:, :, :3]], axis=2)

    idx20f = idx20.reshape(_P * _KMAX)
    pad61 = jnp.zeros((_B, _N, 61), jnp.float32)
    xg2p = jnp.concatenate([xg2, pad61], axis=2).reshape(_P, 128)
    ft4 = _sc_gather_max20(xg2p, idx20f)
    y3 = _gather_conv(xg1, idx5, dist5, dw2, _KMIN)
    y4 = _edge_conv(xg2, ft4.reshape(_B, _N, 128), dist20, dw3, _KMAX)
    z3 = _bn_act(y3.reshape(_B * _N, 64), r2(dg2), r2(db2), 0.2)
    z4 = _bn_act(y4.reshape(_B * _N, 64), r2(dg3), r2(db3), 0.2)
    xg3 = jnp.concatenate([z3.reshape(_B, _N, 64), xg1], axis=2)
    xg4 = jnp.concatenate([z4.reshape(_B, _N, 64), xg2], axis=2)

    xg4t = xg4[:, :, :128].reshape(_P, 128)
    g6 = _sc_gather_max20(xg4t, idx20f)
    ft6 = jnp.concatenate([g6.reshape(_B, _N, 128),
                           ft4.reshape(_B, _N, 128)[:, :, 64:67]], axis=2)
    y5 = _gather_conv(xg3, idx5, dist5, dw4, _KMIN)
    y6 = _edge_conv(xg4, ft6, dist20, dw5, _KMAX)
    x3 = _mix(y5.reshape(_B * _N, 128), r2(dg4), r2(db4),
              y6.reshape(_B * _N, 128), r2(dg5), r2(db5),
              sigma.reshape(1, 1))

    a = _mha(h.reshape(_B, _N, 128), x3.reshape(_B, _N, 128),
             Wq, r2(bq), Wk, r2(bk), Wv, r2(bv), Wo, r2(bo))

    outp = _cew(h, a.reshape(_B * _N, 128), cew[:, :128], cew[:, 128:],
                r2(ceg), r2(ceb), cw4, r2(cb4), r2(g4), r2(b4))

    res = _fattn(f1, f2, outp.reshape(_B, _N, 128), gamma_p.reshape(1, 1))
    return jnp.transpose(res, (0, 2, 1))[..., None]


# speedup vs baseline: 1.1646x; 1.1646x over previous
import functools
import math

import jax
from jax import lax
import jax.numpy as jnp
from jax.experimental import pallas as pl
from jax.experimental.pallas import tpu as pltpu
from jax.experimental.pallas import tpu_sc as plsc

_B, _N = 16, 1024
_KMIN, _KMAX = 5, 20
_HEADS = 4
_EPS = 1e-5
_P = _B * _N
_NW = 32
_PW = _P // _NW


def _sc_gather_max20(fea_pad, idx_flat):
    k, cp, ch = _KMAX, 128, 8
    ng = ch * k // 80
    nsteps = _PW // ch

    mesh = plsc.VectorSubcoreMesh(core_axis_name="c", subcore_axis_name="s")

    @functools.partial(
        pl.kernel, mesh=mesh,
        out_type=jax.ShapeDtypeStruct((_P, cp), jnp.float32),
        scratch_types=[
            pltpu.VMEM((ch * k,), jnp.int32),
            pltpu.VMEM((ch * k,), jnp.int32),
            pltpu.VMEM((ch * k, cp), jnp.float32),
            pltpu.VMEM((ch * k, cp), jnp.float32),
            pltpu.VMEM((ch, cp), jnp.float32),
            pltpu.SemaphoreType.DMA,
            pltpu.SemaphoreType.DMA,
        ],
    )
    def sc_kernel(fea_hbm, idx_hbm, out_hbm, idxA, idxB, rowsA, rowsB, outv,
                  semA, semB):
        cc_ = lax.axis_index("c")
        ss_ = lax.axis_index("s")
        w = ss_ * 2 + cc_
        base = w * _PW

        def issue(t, idxv, rows, sem):
            pt = base + t * ch
            pltpu.sync_copy(idx_hbm.at[pl.ds(pt * k, ch * k)], idxv)
            for gi in range(ng):
                pltpu.async_copy(fea_hbm.at[idxv.at[pl.ds(gi * 80, 80)]],
                                 rows.at[pl.ds(gi * 80, 80)], sem)

        def finish(t, idxv, rows, sem):
            pt = base + t * ch
            for gi in range(ng):
                pltpu.make_async_copy(fea_hbm.at[idxv.at[pl.ds(gi * 80, 80)]],
                                      rows.at[pl.ds(gi * 80, 80)], sem).wait()
            for p in range(ch):
                r0 = p * k
                for c0 in range(cp // 16):
                    sl = pl.ds(c0 * 16, 16)
                    acc = rows[r0, sl]
                    for j in range(1, k):
                        acc = jnp.maximum(acc, rows[r0 + j, sl])
                    outv[p, sl] = acc
            pltpu.sync_copy(outv, out_hbm.at[pl.ds(pt, ch)])

        issue(0, idxA, rowsA, semA)

        def body(i, carry):
            ta = 2 * i
            issue(ta + 1, idxB, rowsB, semB)
            finish(ta, idxA, rowsA, semA)

            @pl.when(i < nsteps // 2 - 1)
            def _():
                issue(ta + 2, idxA, rowsA, semA)

            finish(ta + 1, idxB, rowsB, semB)
            return carry

        lax.fori_loop(0, nsteps // 2, body, 0)

    return sc_kernel(fea_pad, idx_flat)


def _knn_body(x_ref, dist_ref, idx_ref, idxg_ref):
    p = x_ref[0]
    cmask = jax.lax.broadcasted_iota(jnp.int32, (1, 4), 1) < 3
    p3 = jnp.where(cmask, p, 0.0)
    g = jnp.dot(p3, p3.T, preferred_element_type=jnp.float32)
    sq = jnp.sum(p3 * p3, axis=1, keepdims=True)
    d2 = sq + sq.T - 2.0 * g
    d = jnp.sqrt(jnp.clip(d2, 1e-12, None))
    iota = jax.lax.broadcasted_iota(jnp.int32, (_N, _N), 1)
    dcols, icols = [], []
    for _ in range(_KMAX):
        cur = jnp.min(d, axis=1, keepdims=True)
        am = jnp.min(jnp.where(d == cur, iota, _N), axis=1, keepdims=True)
        dcols.append(cur)
        icols.append(am)
        d = jnp.where(iota == am, jnp.inf, d)
    dist_ref[0] = jnp.concatenate(dcols, axis=1)
    icat = jnp.concatenate(icols, axis=1)
    idx_ref[0] = icat
    idxg_ref[0] = icat + pl.program_id(0) * _N


def _knn(xr):
    return pl.pallas_call(
        _knn_body,
        grid=(_B,),
        in_specs=[pl.BlockSpec((1, _N, 4), lambda b: (b, 0, 0))],
        out_specs=[pl.BlockSpec((1, _N, _KMAX), lambda b: (b, 0, 0)),
                   pl.BlockSpec((1, _N, _KMAX), lambda b: (b, 0, 0)),
                   pl.BlockSpec((1, _N, _KMAX), lambda b: (b, 0, 0))],
        out_shape=[jax.ShapeDtypeStruct((_B, _N, _KMAX), jnp.float32),
                   jax.ShapeDtypeStruct((_B, _N, _KMAX), jnp.int32),
                   jax.ShapeDtypeStruct((_B, _N, _KMAX), jnp.int32)],
    )(xr)


def _bn_rows(y, g, b, slope):
    m = jnp.mean(y, axis=0, keepdims=True)
    v = jnp.mean(y * y, axis=0, keepdims=True) - m * m
    z = (y - m) / jnp.sqrt(v + _EPS) * g + b
    if slope == 1.0:
        return z
    return jnp.where(z >= 0, z, slope * z)


def _mlp_body(x_ref, w0, b0r, w1, b1r, w2, b2r, w3, b3r,
              g0, be0, g1, be1, g2, be2, g3, be3, out_ref):
    h = jnp.dot(x_ref[...], w0[...].T, preferred_element_type=jnp.float32) + b0r[...]
    h = _bn_rows(h, g0[...], be0[...], 0.0)
    h = jnp.dot(h, w1[...].T, preferred_element_type=jnp.float32) + b1r[...]
    h = _bn_rows(h, g1[...], be1[...], 0.0)
    h = jnp.dot(h, w2[...].T, preferred_element_type=jnp.float32) + b2r[...]
    h = _bn_rows(h, g2[...], be2[...], 0.0)
    h = jnp.dot(h, w3[...].T, preferred_element_type=jnp.float32) + b3r[...]
    out_ref[...] = _bn_rows(h, g3[...], be3[...], 0.0)


def _mlp(xf, params):
    return pl.pallas_call(
        _mlp_body,
        out_shape=jax.ShapeDtypeStruct((_B * _N, 128), jnp.float32),
    )(xf, *params)


def _gather_conv_body(fea_ref, idx_ref, dist_ref, w_ref, out_ref, *, k):
    f = fea_ref[0]
    idx = idx_ref[0]
    iota = jax.lax.broadcasted_iota(jnp.int32, (_N, _N), 1)
    feat = jnp.full(f.shape, -jnp.inf, f.dtype)
    for j in range(k):
        oh = (idx[:, j:j + 1] == iota).astype(jnp.float32)
        gj = jnp.dot(oh, f, preferred_element_type=jnp.float32)
        feat = jnp.maximum(feat, gj)
    dmax = jnp.max(dist_ref[0], axis=1, keepdims=True)
    edge = jnp.concatenate([f, feat - f, dmax], axis=1)
    out_ref[0] = jnp.dot(edge, w_ref[...].T, preferred_element_type=jnp.float32)


def _gather_conv(fea, idx, dist, w, k):
    c = fea.shape[-1]
    co = w.shape[0]
    return pl.pallas_call(
        functools.partial(_gather_conv_body, k=k),
        grid=(_B,),
        in_specs=[pl.BlockSpec((1, _N, c), lambda b: (b, 0, 0)),
                  pl.BlockSpec((1, _N, k), lambda b: (b, 0, 0)),
                  pl.BlockSpec((1, _N, k), lambda b: (b, 0, 0)),
                  pl.BlockSpec(w.shape, lambda b: (0, 0))],
        out_specs=pl.BlockSpec((1, _N, co), lambda b: (b, 0, 0)),
        out_shape=jax.ShapeDtypeStruct((_B, _N, co), jnp.float32),
    )(fea, idx, dist, w)


def _edge_conv_body(fea_ref, feat_ref, dist_ref, w_ref, out_ref, *, c):
    f = fea_ref[0]
    ft = feat_ref[0][:, :c]
    dmax = jnp.max(dist_ref[0], axis=1, keepdims=True)
    edge = jnp.concatenate([f, ft - f, dmax], axis=1)
    out_ref[0] = jnp.dot(edge, w_ref[...].T, preferred_element_type=jnp.float32)


def _edge_conv(fea, feat, dist, w, k):
    c = fea.shape[-1]
    cp = feat.shape[-1]
    co = w.shape[0]
    return pl.pallas_call(
        functools.partial(_edge_conv_body, c=c),
        grid=(_B,),
        in_specs=[pl.BlockSpec((1, _N, c), lambda b: (b, 0, 0)),
                  pl.BlockSpec((1, _N, cp), lambda b: (b, 0, 0)),
                  pl.BlockSpec((1, _N, k), lambda b: (b, 0, 0)),
                  pl.BlockSpec(w.shape, lambda b: (0, 0))],
        out_specs=pl.BlockSpec((1, _N, co), lambda b: (b, 0, 0)),
        out_shape=jax.ShapeDtypeStruct((_B, _N, co), jnp.float32),
    )(fea, feat, dist, w)


def _gather_rcs_body(fea_ref, idx_ref, dist_ref, w_ref, rw_ref, rb_ref,
                     out_ref, rcs_ref, *, k):
    f = fea_ref[0]
    idx = idx_ref[0]
    iota = jax.lax.broadcasted_iota(jnp.int32, (_N, _N), 1)
    feat = jnp.full(f.shape, -jnp.inf, f.dtype)
    cols = [f[:, 3:4]]
    for j in range(k):
        oh = (idx[:, j:j + 1] == iota).astype(jnp.float32)
        gj = jnp.dot(oh, f, preferred_element_type=jnp.float32)
        feat = jnp.maximum(feat, gj)
        cols.append(gj[:, 3:4])
    rcs = jnp.concatenate(cols, axis=1)
    rcs_ref[0] = jnp.dot(rcs, rw_ref[...].T,
                         preferred_element_type=jnp.float32) + rb_ref[...]
    dmax = jnp.max(dist_ref[0], axis=1, keepdims=True)
    base = f[:, :3]
    edge = jnp.concatenate([base, feat[:, :3] - base, dmax], axis=1)
    out_ref[0] = jnp.dot(edge, w_ref[...].T, preferred_element_type=jnp.float32)


def _gather_rcs(fea, idx, dist, w, rw, rb, k):
    return pl.pallas_call(
        functools.partial(_gather_rcs_body, k=k),
        grid=(_B,),
        in_specs=[pl.BlockSpec((1, _N, 4), lambda b: (b, 0, 0)),
                  pl.BlockSpec((1, _N, k), lambda b: (b, 0, 0)),
                  pl.BlockSpec((1, _N, k), lambda b: (b, 0, 0)),
                  pl.BlockSpec(w.shape, lambda b: (0, 0)),
                  pl.BlockSpec(rw.shape, lambda b: (0, 0)),
                  pl.BlockSpec(rb.shape, lambda b: (0, 0))],
        out_specs=[pl.BlockSpec((1, _N, w.shape[0]), lambda b: (b, 0, 0)),
                   pl.BlockSpec((1, _N, rw.shape[0]), lambda b: (b, 0, 0))],
        out_shape=[jax.ShapeDtypeStruct((_B, _N, w.shape[0]), jnp.float32),
                   jax.ShapeDtypeStruct((_B, _N, rw.shape[0]), jnp.float32)],
    )(fea, idx, dist, w, rw, rb)


def _bn_act_body(y_ref, g_ref, b_ref, o_ref, *, slope):
    o_ref[...] = _bn_rows(y_ref[...], g_ref[...], b_ref[...], slope)


def _bn_act(y, g, b, slope):
    return pl.pallas_call(
        functools.partial(_bn_act_body, slope=slope),
        out_shape=jax.ShapeDtypeStruct(y.shape, jnp.float32),
    )(y, g, b)


def _mix_body(y5_ref, g5, b5, y6_ref, g6, b6, sig_ref, o_ref):
    z5 = _bn_rows(y5_ref[...], g5[...], b5[...], 0.2)
    z6 = _bn_rows(y6_ref[...], g6[...], b6[...], 0.2)
    s = sig_ref[0, 0]
    o_ref[...] = s * z5 + (1.0 - s) * z6


def _mix(y5, g5, b5, y6, g6, b6, sigma):
    return pl.pallas_call(
        _mix_body,
        out_shape=jax.ShapeDtypeStruct(y5.shape, jnp.float32),
    )(y5, g5, b5, y6, g6, b6, sigma)


def _mha_body(h_ref, x3_ref, wq, bq, wk, bk, wv, bv, wo, bo, out_ref):
    hq = h_ref[0]
    ctx = x3_ref[0]
    q = jnp.dot(hq, wq[...].T, preferred_element_type=jnp.float32) + bq[...]
    k_ = jnp.dot(ctx, wk[...].T, preferred_element_type=jnp.float32) + bk[...]
    v = jnp.dot(ctx, wv[...].T, preferred_element_type=jnp.float32) + bv[...]
    dh = 128 // _HEADS
    scale = 1.0 / math.sqrt(1.0 * dh)
    outs = []
    for hh in range(_HEADS):
        qh = q[:, hh * dh:(hh + 1) * dh]
        kh = k_[:, hh * dh:(hh + 1) * dh]
        vh = v[:, hh * dh:(hh + 1) * dh]
        s = jnp.dot(qh, kh.T, preferred_element_type=jnp.float32) * scale
        s = s - jnp.max(s, axis=1, keepdims=True)
        e = jnp.exp(s)
        p = e / jnp.sum(e, axis=1, keepdims=True)
        outs.append(jnp.dot(p, vh, preferred_element_type=jnp.float32))
    o = jnp.concatenate(outs, axis=1)
    out_ref[0] = jnp.dot(o, wo[...].T, preferred_element_type=jnp.float32) + bo[...]


def _mha(h, x3, wq, bq, wk, bk, wv, bv, wo, bo):
    wspec = [pl.BlockSpec(a.shape, lambda b: (0,) * a.ndim)
             for a in (wq, bq, wk, bk, wv, bv, wo, bo)]
    return pl.pallas_call(
        _mha_body,
        grid=(_B,),
        in_specs=[pl.BlockSpec((1, _N, 128), lambda b: (b, 0, 0)),
                  pl.BlockSpec((1, _N, 128), lambda b: (b, 0, 0))] + wspec,
        out_specs=pl.BlockSpec((1, _N, 128), lambda b: (b, 0, 0)),
        out_shape=jax.ShapeDtypeStruct((_B, _N, 128), jnp.float32),
    )(h, x3, wq, bq, wk, bk, wv, bv, wo, bo)


def _cew_body(h_ref, a_ref, wA, wB, ceg, ceb, w4, b4r, g4, be4, out_ref):
    y = (jnp.dot(h_ref[...], wA[...].T, preferred_element_type=jnp.float32)
         + jnp.dot(a_ref[...], wB[...].T, preferred_element_type=jnp.float32))
    x3e = _bn_rows(y, ceg[...], ceb[...], 0.0)
    y2 = jnp.dot(x3e, w4[...].T, preferred_element_type=jnp.float32) + b4r[...]
    out_ref[...] = _bn_rows(y2, g4[...], be4[...], 1.0)


def _cew(h, a, wA, wB, ceg, ceb, w4, b4r, g4, be4):
    return pl.pallas_call(
        _cew_body,
        out_shape=jax.ShapeDtypeStruct((_B * _N, 128), jnp.float32),
    )(h, a, wA, wB, ceg, ceb, w4, b4r, g4, be4)


def _fattn_body(f1_ref, f2_ref, o_ref, gam_ref, out_ref):
    f1 = f1_ref[0]
    f2 = f2_ref[0]
    ob = o_ref[0]
    s = jnp.dot(f1, f2.T, preferred_element_type=jnp.float32) / math.sqrt(32.0)
    s = s - jnp.max(s, axis=1, keepdims=True)
    e = jnp.exp(s)
    p = e / jnp.sum(e, axis=1, keepdims=True)
    enh = jnp.dot(p, ob, preferred_element_type=jnp.float32)
    out_ref[0] = ob + gam_ref[0, 0] * enh


def _fattn(f1, f2, o, gamma):
    return pl.pallas_call(
        _fattn_body,
        grid=(_B,),
        in_specs=[pl.BlockSpec((1, _N, 32), lambda b: (b, 0, 0)),
                  pl.BlockSpec((1, _N, 32), lambda b: (b, 0, 0)),
                  pl.BlockSpec((1, _N, 128), lambda b: (b, 0, 0)),
                  pl.BlockSpec((1, 1), lambda b: (0, 0))],
        out_specs=pl.BlockSpec((1, _N, 128), lambda b: (b, 0, 0)),
        out_shape=jax.ShapeDtypeStruct((_B, _N, 128), jnp.float32),
    )(f1, f2, o, gamma)


def kernel(x, cw0, cb0, cw1, cb1, cw2, cb2, cw3, cb3, cw4, cb4, g0, b0, g1, b1, g2, b2, g3, b3, g4, b4, dw0, dg0, db0, dw1, dg1, db1, dw2, dg2, db2, dw3, dg3, db3, dw4, dg4, db4, dw5, dg5, db5, cew, ceg, ceb, Wq, bq, Wk, bk, Wv, bv, Wo, bo, sigma, gamma_p, rw1, rb1, rw2, rb2):
    xr = x[:, 0]
    xf = xr.reshape(_B * _N, 4)

    r2 = lambda a: a.reshape(1, -1)

    dist20, idx20, idxg20 = _knn(xr)
    dist5, idx5 = dist20[:, :, :_KMIN], idx20[:, :, :_KMIN]

    h = _mlp(xf, (cw0, r2(cb0), cw1, r2(cb1), cw2, r2(cb2), cw3, r2(cb3),
                  r2(g0), r2(b0), r2(g1), r2(b1), r2(g2), r2(b2), r2(g3), r2(b3)))

    y1, f1 = _gather_rcs(xr, idx5, dist5, dw0, rw1, r2(rb1), _KMIN)
    y2, f2 = _gather_rcs(xr, idx20, dist20, dw1, rw2, r2(rb2), _KMAX)
    z1 = _bn_act(y1.reshape(_B * _N, 64), r2(dg0), r2(db0), 0.2)
    z2 = _bn_act(y2.reshape(_B * _N, 64), r2(dg1), r2(db1), 0.2)
    xg1 = jnp.concatenate([z1.reshape(_B, _N, 64), xr[:, :, :3]], axis=2)
    xg2 = jnp.concatenate([z2.reshape(_B, _N, 64), xr[:, :, :3]], axis=2)

    idx20f = idxg20.reshape(_P * _KMAX)
    pad61 = jnp.zeros((_B, _N, 61), jnp.float32)
    xg2p = jnp.concatenate([xg2, pad61], axis=2).reshape(_P, 128)
    ft4 = _sc_gather_max20(xg2p, idx20f)
    y3 = _gather_conv(xg1, idx5, dist5, dw2, _KMIN)
    y4 = _edge_conv(xg2, ft4.reshape(_B, _N, 128), dist20, dw3, _KMAX)
    z3 = _bn_act(y3.reshape(_B * _N, 64), r2(dg2), r2(db2), 0.2)
    z4 = _bn_act(y4.reshape(_B * _N, 64), r2(dg3), r2(db3), 0.2)
    xg3 = jnp.concatenate([z3.reshape(_B, _N, 64), xg1], axis=2)
    xg4 = jnp.concatenate([z4.reshape(_B, _N, 64), xg2], axis=2)

    xg4t = xg4[:, :, :128].reshape(_P, 128)
    g6 = _sc_gather_max20(xg4t, idx20f)
    ft6 = jnp.concatenate([g6.reshape(_B, _N, 128),
                           ft4.reshape(_B, _N, 128)[:, :, 64:67]], axis=2)
    y5 = _gather_conv(xg3, idx5, dist5, dw4, _KMIN)
    y6 = _edge_conv(xg4, ft6, dist20, dw5, _KMAX)
    x3 = _mix(y5.reshape(_B * _N, 128), r2(dg4), r2(db4),
              y6.reshape(_B * _N, 128), r2(dg5), r2(db5),
              sigma.reshape(1, 1))

    a = _mha(h.reshape(_B, _N, 128), x3.reshape(_B, _N, 128),
             Wq, r2(bq), Wk, r2(bk), Wv, r2(bv), Wo, r2(bo))

    outp = _cew(h, a.reshape(_B * _N, 128), cew[:, :128], cew[:, 128:],
                r2(ceg), r2(ceb), cw4, r2(cb4), r2(g4), r2(b4))

    res = _fattn(f1, f2, outp.reshape(_B, _N, 128), gamma_p.reshape(1, 1))
    return jnp.transpose(res, (0, 2, 1))[..., None]


# speedup vs baseline: 1.2474x; 1.0711x over previous
import functools
import math

import jax
from jax import lax
import jax.numpy as jnp
from jax.experimental import pallas as pl
from jax.experimental.pallas import tpu as pltpu
from jax.experimental.pallas import tpu_sc as plsc

_B, _N = 16, 1024
_KMIN, _KMAX = 5, 20
_HEADS = 4
_EPS = 1e-5
_P = _B * _N
_NW = 32
_PW = _P // _NW


def _sc_gather_max20(fea_pad, idx_flat):
    k, cp, ch = _KMAX, 128, 8
    ng = ch * k // 80
    nsteps = _PW // ch

    mesh = plsc.VectorSubcoreMesh(core_axis_name="c", subcore_axis_name="s")

    @functools.partial(
        pl.kernel, mesh=mesh,
        out_type=jax.ShapeDtypeStruct((_P, cp), jnp.float32),
        scratch_types=[
            pltpu.VMEM((ch * k,), jnp.int32),
            pltpu.VMEM((ch * k,), jnp.int32),
            pltpu.VMEM((ch * k, cp), jnp.float32),
            pltpu.VMEM((ch * k, cp), jnp.float32),
            pltpu.VMEM((ch, cp), jnp.float32),
            pltpu.SemaphoreType.DMA,
            pltpu.SemaphoreType.DMA,
        ],
    )
    def sc_kernel(fea_hbm, idx_hbm, out_hbm, idxA, idxB, rowsA, rowsB, outv,
                  semA, semB):
        cc_ = lax.axis_index("c")
        ss_ = lax.axis_index("s")
        w = ss_ * 2 + cc_
        base = w * _PW

        def issue(t, idxv, rows, sem):
            pt = base + t * ch
            pltpu.sync_copy(idx_hbm.at[pl.ds(pt * k, ch * k)], idxv)
            for gi in range(ng):
                pltpu.async_copy(fea_hbm.at[idxv.at[pl.ds(gi * 80, 80)]],
                                 rows.at[pl.ds(gi * 80, 80)], sem)

        def finish(t, idxv, rows, sem):
            pt = base + t * ch
            for gi in range(ng):
                pltpu.make_async_copy(fea_hbm.at[idxv.at[pl.ds(gi * 80, 80)]],
                                      rows.at[pl.ds(gi * 80, 80)], sem).wait()
            for p in range(ch):
                r0 = p * k
                for c0 in range(cp // 16):
                    sl = pl.ds(c0 * 16, 16)
                    acc = rows[r0, sl]
                    for j in range(1, k):
                        acc = jnp.maximum(acc, rows[r0 + j, sl])
                    outv[p, sl] = acc
            pltpu.sync_copy(outv, out_hbm.at[pl.ds(pt, ch)])

        issue(0, idxA, rowsA, semA)

        def body(i, carry):
            ta = 2 * i
            issue(ta + 1, idxB, rowsB, semB)
            finish(ta, idxA, rowsA, semA)

            @pl.when(i < nsteps // 2 - 1)
            def _():
                issue(ta + 2, idxA, rowsA, semA)

            finish(ta + 1, idxB, rowsB, semB)
            return carry

        lax.fori_loop(0, nsteps // 2, body, 0)

    return sc_kernel(fea_pad, idx_flat)


def _knn_body(x_ref, dist_ref, idx_ref, idxg_ref):
    p = x_ref[0]
    cmask = jax.lax.broadcasted_iota(jnp.int32, (1, 4), 1) < 3
    p3 = jnp.where(cmask, p, 0.0)
    g = jnp.dot(p3, p3.T, preferred_element_type=jnp.float32)
    sq = jnp.sum(p3 * p3, axis=1, keepdims=True)
    d2 = sq + sq.T - 2.0 * g
    d = jnp.sqrt(jnp.clip(d2, 1e-12, None))
    iota = jax.lax.broadcasted_iota(jnp.int32, (_N, _N), 1)
    keys = jnp.bitwise_or(
        jnp.bitwise_and(lax.bitcast_convert_type(d, jnp.int32),
                        jnp.int32(-1024)), iota)
    dcols, icols = [], []
    for _ in range(_KMAX):
        m = jnp.min(keys, axis=1, keepdims=True)
        icols.append(jnp.bitwise_and(m, jnp.int32(1023)))
        dcols.append(lax.bitcast_convert_type(
            jnp.bitwise_and(m, jnp.int32(-1024)), jnp.float32))
        keys = jnp.where(keys == m, jnp.int32(0x7FFFFFFF), keys)
    dist_ref[0] = jnp.concatenate(dcols, axis=1)
    icat = jnp.concatenate(icols, axis=1)
    idx_ref[0] = icat
    idxg_ref[0] = icat + pl.program_id(0) * _N


def _knn(xr):
    return pl.pallas_call(
        _knn_body,
        grid=(_B,),
        in_specs=[pl.BlockSpec((1, _N, 4), lambda b: (b, 0, 0))],
        out_specs=[pl.BlockSpec((1, _N, _KMAX), lambda b: (b, 0, 0)),
                   pl.BlockSpec((1, _N, _KMAX), lambda b: (b, 0, 0)),
                   pl.BlockSpec((1, _N, _KMAX), lambda b: (b, 0, 0))],
        out_shape=[jax.ShapeDtypeStruct((_B, _N, _KMAX), jnp.float32),
                   jax.ShapeDtypeStruct((_B, _N, _KMAX), jnp.int32),
                   jax.ShapeDtypeStruct((_B, _N, _KMAX), jnp.int32)],
    )(xr)


def _bn_rows(y, g, b, slope):
    m = jnp.mean(y, axis=0, keepdims=True)
    v = jnp.mean(y * y, axis=0, keepdims=True) - m * m
    z = (y - m) / jnp.sqrt(v + _EPS) * g + b
    if slope == 1.0:
        return z
    return jnp.where(z >= 0, z, slope * z)


def _mlp_body(x_ref, w0, b0r, w1, b1r, w2, b2r, w3, b3r,
              g0, be0, g1, be1, g2, be2, g3, be3, out_ref):
    h = jnp.dot(x_ref[...], w0[...].T, preferred_element_type=jnp.float32) + b0r[...]
    h = _bn_rows(h, g0[...], be0[...], 0.0)
    h = jnp.dot(h, w1[...].T, preferred_element_type=jnp.float32) + b1r[...]
    h = _bn_rows(h, g1[...], be1[...], 0.0)
    h = jnp.dot(h, w2[...].T, preferred_element_type=jnp.float32) + b2r[...]
    h = _bn_rows(h, g2[...], be2[...], 0.0)
    h = jnp.dot(h, w3[...].T, preferred_element_type=jnp.float32) + b3r[...]
    out_ref[...] = _bn_rows(h, g3[...], be3[...], 0.0)


def _mlp(xf, params):
    return pl.pallas_call(
        _mlp_body,
        out_shape=jax.ShapeDtypeStruct((_B * _N, 128), jnp.float32),
    )(xf, *params)


def _gather_conv_body(fea_ref, idx_ref, dist_ref, w_ref, out_ref, *, k):
    f = fea_ref[0]
    idx = idx_ref[0]
    iota = jax.lax.broadcasted_iota(jnp.int32, (_N, _N), 1)
    feat = jnp.full(f.shape, -jnp.inf, f.dtype)
    for j in range(k):
        oh = (idx[:, j:j + 1] == iota).astype(jnp.float32)
        gj = jnp.dot(oh, f, preferred_element_type=jnp.float32)
        feat = jnp.maximum(feat, gj)
    dmax = jnp.max(dist_ref[0], axis=1, keepdims=True)
    edge = jnp.concatenate([f, feat - f, dmax], axis=1)
    out_ref[0] = jnp.dot(edge, w_ref[...].T, preferred_element_type=jnp.float32)


def _gather_conv(fea, idx, dist, w, k):
    c = fea.shape[-1]
    co = w.shape[0]
    return pl.pallas_call(
        functools.partial(_gather_conv_body, k=k),
        grid=(_B,),
        in_specs=[pl.BlockSpec((1, _N, c), lambda b: (b, 0, 0)),
                  pl.BlockSpec((1, _N, k), lambda b: (b, 0, 0)),
                  pl.BlockSpec((1, _N, k), lambda b: (b, 0, 0)),
                  pl.BlockSpec(w.shape, lambda b: (0, 0))],
        out_specs=pl.BlockSpec((1, _N, co), lambda b: (b, 0, 0)),
        out_shape=jax.ShapeDtypeStruct((_B, _N, co), jnp.float32),
    )(fea, idx, dist, w)


def _edge_conv_body(fea_ref, feat_ref, dist_ref, w_ref, out_ref, *, c):
    f = fea_ref[0]
    ft = feat_ref[0][:, :c]
    dmax = jnp.max(dist_ref[0], axis=1, keepdims=True)
    edge = jnp.concatenate([f, ft - f, dmax], axis=1)
    out_ref[0] = jnp.dot(edge, w_ref[...].T, preferred_element_type=jnp.float32)


def _edge_conv(fea, feat, dist, w, k):
    c = fea.shape[-1]
    cp = feat.shape[-1]
    co = w.shape[0]
    return pl.pallas_call(
        functools.partial(_edge_conv_body, c=c),
        grid=(_B,),
        in_specs=[pl.BlockSpec((1, _N, c), lambda b: (b, 0, 0)),
                  pl.BlockSpec((1, _N, cp), lambda b: (b, 0, 0)),
                  pl.BlockSpec((1, _N, k), lambda b: (b, 0, 0)),
                  pl.BlockSpec(w.shape, lambda b: (0, 0))],
        out_specs=pl.BlockSpec((1, _N, co), lambda b: (b, 0, 0)),
        out_shape=jax.ShapeDtypeStruct((_B, _N, co), jnp.float32),
    )(fea, feat, dist, w)


def _gather_rcs_body(fea_ref, idx_ref, dist_ref, w_ref, rw_ref, rb_ref,
                     out_ref, rcs_ref, *, k):
    f = fea_ref[0]
    idx = idx_ref[0]
    iota = jax.lax.broadcasted_iota(jnp.int32, (_N, _N), 1)
    feat = jnp.full(f.shape, -jnp.inf, f.dtype)
    cols = [f[:, 3:4]]
    for j in range(k):
        oh = (idx[:, j:j + 1] == iota).astype(jnp.float32)
        gj = jnp.dot(oh, f, preferred_element_type=jnp.float32)
        feat = jnp.maximum(feat, gj)
        cols.append(gj[:, 3:4])
    rcs = jnp.concatenate(cols, axis=1)
    rcs_ref[0] = jnp.dot(rcs, rw_ref[...].T,
                         preferred_element_type=jnp.float32) + rb_ref[...]
    dmax = jnp.max(dist_ref[0], axis=1, keepdims=True)
    base = f[:, :3]
    edge = jnp.concatenate([base, feat[:, :3] - base, dmax], axis=1)
    out_ref[0] = jnp.dot(edge, w_ref[...].T, preferred_element_type=jnp.float32)


def _gather_rcs(fea, idx, dist, w, rw, rb, k):
    return pl.pallas_call(
        functools.partial(_gather_rcs_body, k=k),
        grid=(_B,),
        in_specs=[pl.BlockSpec((1, _N, 4), lambda b: (b, 0, 0)),
                  pl.BlockSpec((1, _N, k), lambda b: (b, 0, 0)),
                  pl.BlockSpec((1, _N, k), lambda b: (b, 0, 0)),
                  pl.BlockSpec(w.shape, lambda b: (0, 0)),
                  pl.BlockSpec(rw.shape, lambda b: (0, 0)),
                  pl.BlockSpec(rb.shape, lambda b: (0, 0))],
        out_specs=[pl.BlockSpec((1, _N, w.shape[0]), lambda b: (b, 0, 0)),
                   pl.BlockSpec((1, _N, rw.shape[0]), lambda b: (b, 0, 0))],
        out_shape=[jax.ShapeDtypeStruct((_B, _N, w.shape[0]), jnp.float32),
                   jax.ShapeDtypeStruct((_B, _N, rw.shape[0]), jnp.float32)],
    )(fea, idx, dist, w, rw, rb)


def _bn_act_body(y_ref, g_ref, b_ref, o_ref, *, slope):
    o_ref[...] = _bn_rows(y_ref[...], g_ref[...], b_ref[...], slope)


def _bn_act(y, g, b, slope):
    return pl.pallas_call(
        functools.partial(_bn_act_body, slope=slope),
        out_shape=jax.ShapeDtypeStruct(y.shape, jnp.float32),
    )(y, g, b)


def _mix_body(y5_ref, g5, b5, y6_ref, g6, b6, sig_ref, o_ref):
    z5 = _bn_rows(y5_ref[...], g5[...], b5[...], 0.2)
    z6 = _bn_rows(y6_ref[...], g6[...], b6[...], 0.2)
    s = sig_ref[0, 0]
    o_ref[...] = s * z5 + (1.0 - s) * z6


def _mix(y5, g5, b5, y6, g6, b6, sigma):
    return pl.pallas_call(
        _mix_body,
        out_shape=jax.ShapeDtypeStruct(y5.shape, jnp.float32),
    )(y5, g5, b5, y6, g6, b6, sigma)


def _mha_body(h_ref, x3_ref, wq, bq, wk, bk, wv, bv, wo, bo, out_ref):
    hq = h_ref[0]
    ctx = x3_ref[0]
    q = jnp.dot(hq, wq[...].T, preferred_element_type=jnp.float32) + bq[...]
    k_ = jnp.dot(ctx, wk[...].T, preferred_element_type=jnp.float32) + bk[...]
    v = jnp.dot(ctx, wv[...].T, preferred_element_type=jnp.float32) + bv[...]
    dh = 128 // _HEADS
    scale = 1.0 / math.sqrt(1.0 * dh)
    outs = []
    for hh in range(_HEADS):
        qh = q[:, hh * dh:(hh + 1) * dh]
        kh = k_[:, hh * dh:(hh + 1) * dh]
        vh = v[:, hh * dh:(hh + 1) * dh]
        s = jnp.dot(qh, kh.T, preferred_element_type=jnp.float32) * scale
        s = s - jnp.max(s, axis=1, keepdims=True)
        e = jnp.exp(s)
        p = e / jnp.sum(e, axis=1, keepdims=True)
        outs.append(jnp.dot(p, vh, preferred_element_type=jnp.float32))
    o = jnp.concatenate(outs, axis=1)
    out_ref[0] = jnp.dot(o, wo[...].T, preferred_element_type=jnp.float32) + bo[...]


def _mha(h, x3, wq, bq, wk, bk, wv, bv, wo, bo):
    wspec = [pl.BlockSpec(a.shape, lambda b: (0,) * a.ndim)
             for a in (wq, bq, wk, bk, wv, bv, wo, bo)]
    return pl.pallas_call(
        _mha_body,
        grid=(_B,),
        in_specs=[pl.BlockSpec((1, _N, 128), lambda b: (b, 0, 0)),
                  pl.BlockSpec((1, _N, 128), lambda b: (b, 0, 0))] + wspec,
        out_specs=pl.BlockSpec((1, _N, 128), lambda b: (b, 0, 0)),
        out_shape=jax.ShapeDtypeStruct((_B, _N, 128), jnp.float32),
    )(h, x3, wq, bq, wk, bk, wv, bv, wo, bo)


def _cew_body(h_ref, a_ref, wA, wB, ceg, ceb, w4, b4r, g4, be4, out_ref):
    y = (jnp.dot(h_ref[...], wA[...].T, preferred_element_type=jnp.float32)
         + jnp.dot(a_ref[...], wB[...].T, preferred_element_type=jnp.float32))
    x3e = _bn_rows(y, ceg[...], ceb[...], 0.0)
    y2 = jnp.dot(x3e, w4[...].T, preferred_element_type=jnp.float32) + b4r[...]
    out_ref[...] = _bn_rows(y2, g4[...], be4[...], 1.0)


def _cew(h, a, wA, wB, ceg, ceb, w4, b4r, g4, be4):
    return pl.pallas_call(
        _cew_body,
        out_shape=jax.ShapeDtypeStruct((_B * _N, 128), jnp.float32),
    )(h, a, wA, wB, ceg, ceb, w4, b4r, g4, be4)


def _fattn_body(f1_ref, f2_ref, o_ref, gam_ref, out_ref):
    f1 = f1_ref[0]
    f2 = f2_ref[0]
    ob = o_ref[0]
    s = jnp.dot(f1, f2.T, preferred_element_type=jnp.float32) / math.sqrt(32.0)
    s = s - jnp.max(s, axis=1, keepdims=True)
    e = jnp.exp(s)
    p = e / jnp.sum(e, axis=1, keepdims=True)
    enh = jnp.dot(p, ob, preferred_element_type=jnp.float32)
    out_ref[0] = ob + gam_ref[0, 0] * enh


def _fattn(f1, f2, o, gamma):
    return pl.pallas_call(
        _fattn_body,
        grid=(_B,),
        in_specs=[pl.BlockSpec((1, _N, 32), lambda b: (b, 0, 0)),
                  pl.BlockSpec((1, _N, 32), lambda b: (b, 0, 0)),
                  pl.BlockSpec((1, _N, 128), lambda b: (b, 0, 0)),
                  pl.BlockSpec((1, 1), lambda b: (0, 0))],
        out_specs=pl.BlockSpec((1, _N, 128), lambda b: (b, 0, 0)),
        out_shape=jax.ShapeDtypeStruct((_B, _N, 128), jnp.float32),
    )(f1, f2, o, gamma)


def kernel(x, cw0, cb0, cw1, cb1, cw2, cb2, cw3, cb3, cw4, cb4, g0, b0, g1, b1, g2, b2, g3, b3, g4, b4, dw0, dg0, db0, dw1, dg1, db1, dw2, dg2, db2, dw3, dg3, db3, dw4, dg4, db4, dw5, dg5, db5, cew, ceg, ceb, Wq, bq, Wk, bk, Wv, bv, Wo, bo, sigma, gamma_p, rw1, rb1, rw2, rb2):
    xr = x[:, 0]
    xf = xr.reshape(_B * _N, 4)

    r2 = lambda a: a.reshape(1, -1)

    dist20, idx20, idxg20 = _knn(xr)
    dist5, idx5 = dist20[:, :, :_KMIN], idx20[:, :, :_KMIN]

    h = _mlp(xf, (cw0, r2(cb0), cw1, r2(cb1), cw2, r2(cb2), cw3, r2(cb3),
                  r2(g0), r2(b0), r2(g1), r2(b1), r2(g2), r2(b2), r2(g3), r2(b3)))

    y1, f1 = _gather_rcs(xr, idx5, dist5, dw0, rw1, r2(rb1), _KMIN)
    y2, f2 = _gather_rcs(xr, idx20, dist20, dw1, rw2, r2(rb2), _KMAX)
    z1 = _bn_act(y1.reshape(_B * _N, 64), r2(dg0), r2(db0), 0.2)
    z2 = _bn_act(y2.reshape(_B * _N, 64), r2(dg1), r2(db1), 0.2)
    xg1 = jnp.concatenate([z1.reshape(_B, _N, 64), xr[:, :, :3]], axis=2)
    xg2 = jnp.concatenate([z2.reshape(_B, _N, 64), xr[:, :, :3]], axis=2)

    idx20f = idxg20.reshape(_P * _KMAX)
    pad61 = jnp.zeros((_B, _N, 61), jnp.float32)
    xg2p = jnp.concatenate([xg2, pad61], axis=2).reshape(_P, 128)
    ft4 = _sc_gather_max20(xg2p, idx20f)
    y3 = _gather_conv(xg1, idx5, dist5, dw2, _KMIN)
    y4 = _edge_conv(xg2, ft4.reshape(_B, _N, 128), dist20, dw3, _KMAX)
    z3 = _bn_act(y3.reshape(_B * _N, 64), r2(dg2), r2(db2), 0.2)
    z4 = _bn_act(y4.reshape(_B * _N, 64), r2(dg3), r2(db3), 0.2)
    xg3 = jnp.concatenate([z3.reshape(_B, _N, 64), xg1], axis=2)
    xg4 = jnp.concatenate([z4.reshape(_B, _N, 64), xg2], axis=2)

    xg4t = xg4[:, :, :128].reshape(_P, 128)
    g6 = _sc_gather_max20(xg4t, idx20f)
    ft6 = jnp.concatenate([g6.reshape(_B, _N, 128),
                           ft4.reshape(_B, _N, 128)[:, :, 64:67]], axis=2)
    y5 = _gather_conv(xg3, idx5, dist5, dw4, _KMIN)
    y6 = _edge_conv(xg4, ft6, dist20, dw5, _KMAX)
    x3 = _mix(y5.reshape(_B * _N, 128), r2(dg4), r2(db4),
              y6.reshape(_B * _N, 128), r2(dg5), r2(db5),
              sigma.reshape(1, 1))

    a = _mha(h.reshape(_B, _N, 128), x3.reshape(_B, _N, 128),
             Wq, r2(bq), Wk, r2(bk), Wv, r2(bv), Wo, r2(bo))

    outp = _cew(h, a.reshape(_B * _N, 128), cew[:, :128], cew[:, 128:],
                r2(ceg), r2(ceb), cw4, r2(cb4), r2(g4), r2(b4))

    res = _fattn(f1, f2, outp.reshape(_B, _N, 128), gamma_p.reshape(1, 1))
    return jnp.transpose(res, (0, 2, 1))[..., None]
